# Initial kernel scaffold; baseline (speedup 1.0000x reference)
import jax
import jax.numpy as jnp
from jax.experimental import pallas as pl


def kernel(x_main, edge_index_main, edge_weight_main, x_aux, edge_index_aux, edge_weight_aux, W1m, b1m, W2m, b2m, W1a, b1a, W2a, b2a, Wr, br):
    raise NotImplementedError("write your pallas kernel here")


# speedup vs baseline: 7.1787x; 7.1787x over previous
import functools

import jax
import jax.numpy as jnp
from jax import lax
from jax.experimental import pallas as pl
from jax.experimental.pallas import tpu as pltpu
from jax.experimental.pallas import tpu_sc as plsc

F32 = jnp.float32
I32 = jnp.int32

NC = 2
NS = 16
NW = NC * NS
KCH = 128


def _agg_body(n, e_per_w, n_chunks, with_gather, *refs):
    if with_gather:
        (g_hbm, r_hbm, c_hbm, w_hbm, out_hbm,
         ridx, cidx, wbuf, rows, acc_sh, sem) = refs
    else:
        (c_hbm, w_hbm, out_hbm,
         cidx, wbuf, rows, acc_sh, sem) = refs
        r_hbm = g_hbm = ridx = None
    cid = lax.axis_index("c")
    sid = lax.axis_index("s")
    wid = cid * NS + sid
    rows_per_tile = n // NS
    for j in range(KCH):
        rows[j, :] = jnp.zeros((16,), F32)
    n_sub = rows_per_tile // KCH
    for t in range(n_sub):
        pltpu.sync_copy(rows,
                        acc_sh.at[pl.ds(sid * rows_per_tile + t * KCH, KCH)])
    plsc.subcore_barrier()

    def chunk(i, carry):
        base = wid * e_per_w + i * KCH
        pltpu.sync_copy(c_hbm.at[pl.ds(wid * n_chunks + i, 1)], cidx)
        pltpu.sync_copy(w_hbm.at[pl.ds(base, KCH)], wbuf.at[pl.ds(16, KCH)])
        if with_gather:
            pltpu.sync_copy(r_hbm.at[pl.ds(base, KCH)], ridx)
            pltpu.async_copy(g_hbm.at[ridx], rows, sem).wait()
            for j in range(KCH):
                wv = plsc.load_gather(wbuf, [jnp.full((16,), j + 16, I32)])
                rows[j, :] = rows[j, :] * wv
        else:
            for j in range(KCH):
                wv = plsc.load_gather(wbuf, [jnp.full((16,), j + 16, I32)])
                rows[j, :] = wv
        pltpu.sync_copy(rows, acc_sh.at[cidx.at[0]], add=True)
        return carry

    lax.fori_loop(0, n_chunks, chunk, 0)
    plsc.subcore_barrier()
    for t in range(n_sub):
        sl = pl.ds(sid * rows_per_tile + t * KCH, KCH)
        pltpu.sync_copy(acc_sh.at[sl], out_hbm.at[cid, sl])


def _make_agg(n_acc, e_pad, with_gather):
    n = n_acc
    e_per_w = e_pad // NW
    n_chunks = e_per_w // KCH
    mesh = plsc.VectorSubcoreMesh(core_axis_name="c", subcore_axis_name="s",
                                  num_cores=NC, num_subcores=NS)
    scratch = []
    if with_gather:
        scratch.append(pltpu.VMEM((KCH,), I32))
    scratch += [
        pltpu.VMEM((1, KCH), I32),
        pltpu.VMEM((KCH + 16,), F32),
        pltpu.VMEM((KCH, 16), F32),
        pltpu.VMEM_SHARED((n, 16), F32),
        pltpu.SemaphoreType.DMA,
    ]
    body = functools.partial(_agg_body, n, e_per_w, n_chunks, with_gather)
    return pl.kernel(
        body,
        out_type=jax.ShapeDtypeStruct((NC, n, 16), F32),
        mesh=mesh,
        scratch_types=scratch,
        compiler_params=pltpu.CompilerParams(needs_layout_passes=False,
                                             use_tc_tiling_on_sc=False),
    )


def _prep_body(dm_ref, da_ref, xm_ref, xa_ref, W1m_ref, W2m_ref, W1a_ref,
               W2a_ref, b1m_ref, b1a_ref,
               um_ref, basem_ref, dinvm_ref, ua_ref, basea_ref, dinva_ref):
    for (d_ref, x_ref, W1_ref, W2_ref, b1_ref, u_ref, base_ref, dinv_ref) in (
        (dm_ref, xm_ref, W1m_ref, W2m_ref, b1m_ref, um_ref, basem_ref, dinvm_ref),
        (da_ref, xa_ref, W1a_ref, W2a_ref, b1a_ref, ua_ref, basea_ref, dinva_ref),
    ):
        W12 = jnp.dot(W1_ref[...], W2_ref[...], preferred_element_type=F32)
        bb = jnp.dot(b1_ref[...], W2_ref[...], preferred_element_type=F32)
        deg = 1.0 + d_ref[0] + d_ref[1]
        dinv = lax.rsqrt(deg)
        q = jnp.dot(x_ref[...], W12, preferred_element_type=F32)
        u = dinv * q
        u_ref[...] = u
        base_ref[...] = dinv * u + bb
        dinv_ref[...] = dinv


def _mid_body(a1m_ref, a1a_ref, dinvm_ref, dinva_ref, basem_ref, basea_ref,
              b2m_ref, b2a_ref, g2m_ref, base2m_ref, g2a_ref, base2a_ref):
    for (a_ref, dinv_ref, base_ref, b2_ref, g2_ref, base2_ref) in (
        (a1m_ref, dinvm_ref, basem_ref, b2m_ref, g2m_ref, base2m_ref),
        (a1a_ref, dinva_ref, basea_ref, b2a_ref, g2a_ref, base2a_ref),
    ):
        dinv = dinv_ref[...]
        h2 = dinv * (a_ref[0] + a_ref[1]) + base_ref[...]
        g2 = dinv * h2
        g2_ref[...] = g2
        base2_ref[...] = dinv * g2 + b2_ref[...]


def _final_body(a2m_ref, a2a_ref, dinvm_ref, dinva_ref, base2m_ref,
                base2a_ref, hm_ref, ha_ref):
    hm_ref[...] = dinvm_ref[...] * (a2m_ref[0] + a2m_ref[1]) + base2m_ref[...]
    ha_ref[...] = dinva_ref[...] * (a2a_ref[0] + a2a_ref[1]) + base2a_ref[...]


def _summary_body(wr_ref, hm_ref, ha_ref, br_ref, sm_ref, sa_ref, *, n_steps):
    i = pl.program_id(0)

    @pl.when(i == 0)
    def _():
        sm_ref[...] = jnp.zeros_like(sm_ref)
        sa_ref[...] = jnp.zeros_like(sa_ref)

    dn = (((1,), (1,)), ((), ()))
    sm_ref[...] += lax.dot_general(hm_ref[...], wr_ref[...], dn,
                                   preferred_element_type=F32)
    sa_ref[...] += lax.dot_general(ha_ref[...], wr_ref[...], dn,
                                   preferred_element_type=F32)

    @pl.when(i == n_steps - 1)
    def _():
        sm_ref[...] = jnp.maximum(sm_ref[...] + br_ref[...], 0.0)
        sa_ref[...] = jnp.maximum(sa_ref[...] + br_ref[...], 0.0)


def _decode_body(hi_ref, hj_ref, out_ref):
    dn = (((1,), (1,)), ((), ()))
    z = lax.dot_general(hi_ref[...], hj_ref[...], dn,
                        preferred_element_type=F32)
    out_ref[...] = jax.nn.sigmoid(z)


def _run_decode(h, n):
    bm, bn = 1024, 1024
    gm, gn = pl.cdiv(n, bm), pl.cdiv(n, bn)
    return pl.pallas_call(
        _decode_body,
        grid=(gm, gn),
        in_specs=[
            pl.BlockSpec((bm, 16), lambda i, j: (i, 0)),
            pl.BlockSpec((bn, 16), lambda i, j: (j, 0)),
        ],
        out_specs=pl.BlockSpec((bm, bn), lambda i, j: (i, j)),
        out_shape=jax.ShapeDtypeStruct((n, n), F32),
    )(h, h)


def kernel(x_main, edge_index_main, edge_weight_main,
           x_aux, edge_index_aux, edge_weight_aux,
           W1m, b1m, W2m, b2m, W1a, b1a, W2a, b2a, Wr, br):
    n, d_in = x_main.shape
    e = edge_index_main.shape[1]
    d_lat = W2m.shape[1]
    e_per_w = pl.cdiv(pl.cdiv(e, NW), KCH) * KCH
    e_pad = e_per_w * NW
    pad = e_pad - e

    def pad_edges(ei, ew):
        r = jnp.concatenate([ei[0], jnp.zeros((pad,), I32)])
        c = jnp.concatenate([ei[1], jnp.zeros((pad,), I32)])
        w = jnp.concatenate([ew, jnp.zeros((pad,), F32)])
        return r, c.reshape(e_pad // KCH, KCH), w

    rm, cm2, wm = pad_edges(edge_index_main, edge_weight_main)
    ra, ca2, wa = pad_edges(edge_index_aux, edge_weight_aux)

    n_acc = pl.cdiv(n, NS * KCH) * NS * KCH
    deg_call = _make_agg(n_acc, e_pad, with_gather=False)
    agg_call = _make_agg(n_acc, e_pad, with_gather=True)

    dm = deg_call(cm2, wm)
    da = deg_call(ca2, wa)

    bn = 2000
    grid = (n // bn,)
    full16 = pl.BlockSpec((2, bn, d_lat), lambda i: (0, i, 0))
    blk16 = pl.BlockSpec((bn, d_lat), lambda i: (i, 0))
    blkx = pl.BlockSpec((bn, d_in), lambda i: (i, 0))
    w1spec = pl.BlockSpec((d_in, d_in), lambda i: (0, 0))
    w2spec = pl.BlockSpec((d_in, d_lat), lambda i: (0, 0))
    bspec = pl.BlockSpec((1, d_in), lambda i: (0, 0))
    st16 = jax.ShapeDtypeStruct((n, d_lat), F32)
    um, basem, dinvm, ua, basea, dinva = pl.pallas_call(
        _prep_body,
        grid=grid,
        in_specs=[full16, full16, blkx, blkx, w1spec, w2spec, w1spec, w2spec,
                  bspec, bspec],
        out_specs=[blk16] * 6,
        out_shape=[st16] * 6,
    )(dm, da, x_main, x_aux, W1m, W2m, W1a, W2a,
      b1m.reshape(1, d_in), b1a.reshape(1, d_in))

    a1m = agg_call(um, rm, cm2, wm)
    a1a = agg_call(ua, ra, ca2, wa)

    b2spec = pl.BlockSpec((1, d_lat), lambda i: (0, 0))
    g2m, base2m, g2a, base2a = pl.pallas_call(
        _mid_body,
        grid=grid,
        in_specs=[full16, full16, blk16, blk16, blk16, blk16, b2spec, b2spec],
        out_specs=[blk16] * 4,
        out_shape=[st16] * 4,
    )(a1m, a1a, dinvm, dinva, basem, basea,
      b2m.reshape(1, d_lat), b2a.reshape(1, d_lat))

    a2m = agg_call(g2m, rm, cm2, wm)
    a2a = agg_call(g2a, ra, ca2, wa)

    hm, ha = pl.pallas_call(
        _final_body,
        grid=grid,
        in_specs=[full16, full16, blk16, blk16, blk16, blk16],
        out_specs=[blk16] * 2,
        out_shape=[st16] * 2,
    )(a2m, a2a, dinvm, dinva, base2m, base2a)

    flat = n * d_lat
    kb = 16000
    n_steps = flat // kb
    sm, sa = pl.pallas_call(
        functools.partial(_summary_body, n_steps=n_steps),
        grid=(n_steps,),
        in_specs=[
            pl.BlockSpec((d_lat, kb), lambda i: (0, i)),
            pl.BlockSpec((1, kb), lambda i: (0, i)),
            pl.BlockSpec((1, kb), lambda i: (0, i)),
            pl.BlockSpec((1, d_lat), lambda i: (0, 0)),
        ],
        out_specs=[pl.BlockSpec((1, d_lat), lambda i: (0, 0))] * 2,
        out_shape=[jax.ShapeDtypeStruct((1, d_lat), F32)] * 2,
    )(Wr, hm.reshape(1, flat), ha.reshape(1, flat), br.reshape(1, d_lat))

    decode_main = _run_decode(hm, n)
    decode_aux = _run_decode(ha, n)

    return (hm, ha, sm.reshape(d_lat), sa.reshape(d_lat),
            decode_main, decode_aux)


# speedup vs baseline: 8.8028x; 1.2262x over previous
import functools

import jax
import jax.numpy as jnp
from jax import lax
from jax.experimental import pallas as pl
from jax.experimental.pallas import tpu as pltpu
from jax.experimental.pallas import tpu_sc as plsc

F32 = jnp.float32
I32 = jnp.int32

NC = 2
NS = 16
NW = NC * NS
KCH = 128


NBUF = 4


def _zero_acc(rows0, acc_sh, sid, rows_per_tile):
    for j in range(KCH):
        rows0[j, :] = jnp.zeros((16,), F32)
    for t in range(rows_per_tile // KCH):
        pltpu.sync_copy(rows0,
                        acc_sh.at[pl.ds(sid * rows_per_tile + t * KCH, KCH)])


def _agg_one_graph(wid, n_chunks, with_gather, g_hbm, r_hbm, c_hbm, w_hbm,
                   ridx, cidx, wbuf, rowsb, acc_sh, sems):
    pltpu.sync_copy(c_hbm.at[pl.ds(wid * n_chunks, n_chunks)], cidx)
    pltpu.sync_copy(w_hbm.at[pl.ds(wid * n_chunks * KCH, n_chunks * KCH)],
                    wbuf.at[pl.ds(16, n_chunks * KCH)])
    if with_gather:
        pltpu.sync_copy(r_hbm.at[pl.ds(wid * n_chunks * KCH, n_chunks * KCH)],
                        ridx)
        for b in range(NBUF):
            pltpu.async_copy(g_hbm.at[ridx.at[pl.ds(b * KCH, KCH)]],
                             rowsb[b], sems[b])

    def chunk_block(i, carry):
        for b in range(NBUF):
            chunk = i * NBUF + b
            wb16 = chunk * KCH + 16
            rows = rowsb[b]
            if with_gather:
                pltpu.make_async_copy(g_hbm.at[ridx.at[pl.ds(b * KCH, KCH)]],
                                      rows, sems[b]).wait()
                for j in range(KCH):
                    wv = plsc.load_gather(wbuf, [jnp.full((16,), wb16 + j, I32)])
                    rows[j, :] = rows[j, :] * wv
            else:
                for j in range(KCH):
                    wv = plsc.load_gather(wbuf, [jnp.full((16,), wb16 + j, I32)])
                    rows[j, :] = wv
            pltpu.sync_copy(rows, acc_sh.at[cidx.at[chunk]], add=True)
            if with_gather:
                nxt = chunk + NBUF

                @pl.when(nxt < n_chunks)
                def _():
                    pltpu.async_copy(
                        g_hbm.at[ridx.at[pl.ds(nxt * KCH, KCH)]],
                        rows, sems[b])
        return carry

    lax.fori_loop(0, n_chunks // NBUF, chunk_block, 0)


def _agg_body(n, e_per_w, n_chunks, with_gather, *refs):
    if with_gather:
        (gm_hbm, rm_hbm, cm_hbm, wm_hbm, ga_hbm, ra_hbm, ca_hbm, wa_hbm,
         outm_hbm, outa_hbm, ridx, cidx, wbuf, r0, r1, r2, r3,
         accm_sh, acca_sh, s0, s1, s2, s3) = refs
    else:
        (cm_hbm, wm_hbm, ca_hbm, wa_hbm, outm_hbm, outa_hbm,
         cidx, wbuf, r0, r1, r2, r3, accm_sh, acca_sh, s0, s1, s2, s3) = refs
        gm_hbm = rm_hbm = ga_hbm = ra_hbm = ridx = None
    rowsb = [r0, r1, r2, r3]
    sems = [s0, s1, s2, s3]
    cid = lax.axis_index("c")
    sid = lax.axis_index("s")
    wid = cid * NS + sid
    rows_per_tile = n // NS
    _zero_acc(r0, accm_sh, sid, rows_per_tile)
    _zero_acc(r0, acca_sh, sid, rows_per_tile)
    plsc.subcore_barrier()
    _agg_one_graph(wid, n_chunks, with_gather, gm_hbm, rm_hbm, cm_hbm, wm_hbm,
                   ridx, cidx, wbuf, rowsb, accm_sh, sems)
    _agg_one_graph(wid, n_chunks, with_gather, ga_hbm, ra_hbm, ca_hbm, wa_hbm,
                   ridx, cidx, wbuf, rowsb, acca_sh, sems)
    plsc.subcore_barrier()
    for t in range(rows_per_tile // KCH):
        sl = pl.ds(sid * rows_per_tile + t * KCH, KCH)
        pltpu.sync_copy(accm_sh.at[sl], outm_hbm.at[cid, sl])
        pltpu.sync_copy(acca_sh.at[sl], outa_hbm.at[cid, sl])


def _make_agg(n_acc, e_pad, with_gather):
    n = n_acc
    e_per_w = e_pad // NW
    n_chunks = e_per_w // KCH
    mesh = plsc.VectorSubcoreMesh(core_axis_name="c", subcore_axis_name="s",
                                  num_cores=NC, num_subcores=NS)
    scratch = []
    if with_gather:
        scratch.append(pltpu.VMEM((e_per_w,), I32))
    scratch += [
        pltpu.VMEM((n_chunks, KCH), I32),
        pltpu.VMEM((e_per_w + 16,), F32),
        pltpu.VMEM((KCH, 16), F32),
        pltpu.VMEM((KCH, 16), F32),
        pltpu.VMEM((KCH, 16), F32),
        pltpu.VMEM((KCH, 16), F32),
        pltpu.VMEM_SHARED((n, 16), F32),
        pltpu.VMEM_SHARED((n, 16), F32),
        pltpu.SemaphoreType.DMA,
        pltpu.SemaphoreType.DMA,
        pltpu.SemaphoreType.DMA,
        pltpu.SemaphoreType.DMA,
    ]
    body = functools.partial(_agg_body, n, e_per_w, n_chunks, with_gather)
    st = jax.ShapeDtypeStruct((NC, n, 16), F32)
    return pl.kernel(
        body,
        out_type=(st, st),
        mesh=mesh,
        scratch_types=scratch,
        compiler_params=pltpu.CompilerParams(needs_layout_passes=False,
                                             use_tc_tiling_on_sc=False),
    )


def _prep_body(dm_ref, da_ref, xm_ref, xa_ref, W1m_ref, W2m_ref, W1a_ref,
               W2a_ref, b1m_ref, b1a_ref,
               um_ref, basem_ref, dinvm_ref, ua_ref, basea_ref, dinva_ref):
    for (d_ref, x_ref, W1_ref, W2_ref, b1_ref, u_ref, base_ref, dinv_ref) in (
        (dm_ref, xm_ref, W1m_ref, W2m_ref, b1m_ref, um_ref, basem_ref, dinvm_ref),
        (da_ref, xa_ref, W1a_ref, W2a_ref, b1a_ref, ua_ref, basea_ref, dinva_ref),
    ):
        bb = jnp.dot(b1_ref[...], W2_ref[...], preferred_element_type=F32)
        deg = 1.0 + d_ref[0] + d_ref[1]
        dinv = 1.0 / jnp.sqrt(deg)
        h1 = jnp.dot(x_ref[...], W1_ref[...], preferred_element_type=F32)
        q = jnp.dot(h1, W2_ref[...], preferred_element_type=F32)
        u = dinv * q
        u_ref[...] = u
        base_ref[...] = dinv * u + bb
        dinv_ref[...] = dinv


def _mid_body(a1m_ref, a1a_ref, dinvm_ref, dinva_ref, basem_ref, basea_ref,
              b2m_ref, b2a_ref, g2m_ref, base2m_ref, g2a_ref, base2a_ref):
    for (a_ref, dinv_ref, base_ref, b2_ref, g2_ref, base2_ref) in (
        (a1m_ref, dinvm_ref, basem_ref, b2m_ref, g2m_ref, base2m_ref),
        (a1a_ref, dinva_ref, basea_ref, b2a_ref, g2a_ref, base2a_ref),
    ):
        dinv = dinv_ref[...]
        h2 = dinv * (a_ref[0] + a_ref[1]) + base_ref[...]
        g2 = dinv * h2
        g2_ref[...] = g2
        base2_ref[...] = dinv * g2 + b2_ref[...]


def _final_body(a2m_ref, a2a_ref, dinvm_ref, dinva_ref, base2m_ref,
                base2a_ref, hm_ref, ha_ref):
    hm_ref[...] = dinvm_ref[...] * (a2m_ref[0] + a2m_ref[1]) + base2m_ref[...]
    ha_ref[...] = dinva_ref[...] * (a2a_ref[0] + a2a_ref[1]) + base2a_ref[...]


def _summary_body(wr_ref, hm_ref, ha_ref, br_ref, sm_ref, sa_ref, *, n_steps):
    i = pl.program_id(0)

    @pl.when(i == 0)
    def _():
        sm_ref[...] = jnp.zeros_like(sm_ref)
        sa_ref[...] = jnp.zeros_like(sa_ref)

    dn = (((1,), (1,)), ((), ()))
    sm_ref[...] += lax.dot_general(hm_ref[...], wr_ref[...], dn,
                                   preferred_element_type=F32)
    sa_ref[...] += lax.dot_general(ha_ref[...], wr_ref[...], dn,
                                   preferred_element_type=F32)

    @pl.when(i == n_steps - 1)
    def _():
        sm_ref[...] = jnp.maximum(sm_ref[...] + br_ref[...], 0.0)
        sa_ref[...] = jnp.maximum(sa_ref[...] + br_ref[...], 0.0)


def _decode_body(hi_ref, hj_ref, out_ref):
    dn = (((1,), (1,)), ((), ()))
    z = lax.dot_general(hi_ref[...], hj_ref[...], dn,
                        preferred_element_type=F32)
    out_ref[...] = jax.nn.sigmoid(z)


def _run_decode(h, n):
    bm, bn = 1024, 1024
    gm, gn = pl.cdiv(n, bm), pl.cdiv(n, bn)
    return pl.pallas_call(
        _decode_body,
        grid=(gm, gn),
        in_specs=[
            pl.BlockSpec((bm, 16), lambda i, j: (i, 0)),
            pl.BlockSpec((bn, 16), lambda i, j: (j, 0)),
        ],
        out_specs=pl.BlockSpec((bm, bn), lambda i, j: (i, j)),
        out_shape=jax.ShapeDtypeStruct((n, n), F32),
    )(h, h)


def kernel(x_main, edge_index_main, edge_weight_main,
           x_aux, edge_index_aux, edge_weight_aux,
           W1m, b1m, W2m, b2m, W1a, b1a, W2a, b2a, Wr, br):
    n, d_in = x_main.shape
    e = edge_index_main.shape[1]
    d_lat = W2m.shape[1]
    e_per_w = pl.cdiv(pl.cdiv(e, NW), KCH * NBUF) * KCH * NBUF
    e_pad = e_per_w * NW
    pad = e_pad - e

    def pad_edges(ei, ew):
        r = jnp.concatenate([ei[0], jnp.zeros((pad,), I32)])
        c = jnp.concatenate([ei[1], jnp.zeros((pad,), I32)])
        w = jnp.concatenate([ew, jnp.zeros((pad,), F32)])
        return r, c.reshape(e_pad // KCH, KCH), w

    rm, cm2, wm = pad_edges(edge_index_main, edge_weight_main)
    ra, ca2, wa = pad_edges(edge_index_aux, edge_weight_aux)

    n_acc = pl.cdiv(n, NS * KCH) * NS * KCH
    deg_call = _make_agg(n_acc, e_pad, with_gather=False)
    agg_call = _make_agg(n_acc, e_pad, with_gather=True)

    dm, da = deg_call(cm2, wm, ca2, wa)

    bn = 2000
    grid = (n // bn,)
    full16 = pl.BlockSpec((2, bn, d_lat), lambda i: (0, i, 0))
    blk16 = pl.BlockSpec((bn, d_lat), lambda i: (i, 0))
    blkx = pl.BlockSpec((bn, d_in), lambda i: (i, 0))
    w1spec = pl.BlockSpec((d_in, d_in), lambda i: (0, 0))
    w2spec = pl.BlockSpec((d_in, d_lat), lambda i: (0, 0))
    bspec = pl.BlockSpec((1, d_in), lambda i: (0, 0))
    st16 = jax.ShapeDtypeStruct((n, d_lat), F32)
    um, basem, dinvm, ua, basea, dinva = pl.pallas_call(
        _prep_body,
        grid=grid,
        in_specs=[full16, full16, blkx, blkx, w1spec, w2spec, w1spec, w2spec,
                  bspec, bspec],
        out_specs=[blk16] * 6,
        out_shape=[st16] * 6,
    )(dm, da, x_main, x_aux, W1m, W2m, W1a, W2a,
      b1m.reshape(1, d_in), b1a.reshape(1, d_in))

    a1m, a1a = agg_call(um, rm, cm2, wm, ua, ra, ca2, wa)

    b2spec = pl.BlockSpec((1, d_lat), lambda i: (0, 0))
    g2m, base2m, g2a, base2a = pl.pallas_call(
        _mid_body,
        grid=grid,
        in_specs=[full16, full16, blk16, blk16, blk16, blk16, b2spec, b2spec],
        out_specs=[blk16] * 4,
        out_shape=[st16] * 4,
    )(a1m, a1a, dinvm, dinva, basem, basea,
      b2m.reshape(1, d_lat), b2a.reshape(1, d_lat))

    a2m, a2a = agg_call(g2m, rm, cm2, wm, g2a, ra, ca2, wa)

    hm, ha = pl.pallas_call(
        _final_body,
        grid=grid,
        in_specs=[full16, full16, blk16, blk16, blk16, blk16],
        out_specs=[blk16] * 2,
        out_shape=[st16] * 2,
    )(a2m, a2a, dinvm, dinva, base2m, base2a)

    flat = n * d_lat
    kb = 16000
    n_steps = flat // kb
    sm, sa = pl.pallas_call(
        functools.partial(_summary_body, n_steps=n_steps),
        grid=(n_steps,),
        in_specs=[
            pl.BlockSpec((d_lat, kb), lambda i: (0, i)),
            pl.BlockSpec((1, kb), lambda i: (0, i)),
            pl.BlockSpec((1, kb), lambda i: (0, i)),
            pl.BlockSpec((1, d_lat), lambda i: (0, 0)),
        ],
        out_specs=[pl.BlockSpec((1, d_lat), lambda i: (0, 0))] * 2,
        out_shape=[jax.ShapeDtypeStruct((1, d_lat), F32)] * 2,
    )(Wr, hm.reshape(1, flat), ha.reshape(1, flat), br.reshape(1, d_lat))

    decode_main = _run_decode(hm, n)
    decode_aux = _run_decode(ha, n)

    return (hm, ha, sm.reshape(d_lat), sa.reshape(d_lat),
            decode_main, decode_aux)
